# Initial kernel scaffold; baseline (speedup 1.0000x reference)
import functools

import jax
import jax.numpy as jnp
from jax import lax
from jax.experimental import pallas as pl
from jax.experimental.pallas import tpu as pltpu
from jax.experimental.pallas import tpu_sc as plsc

HID = 64
NC, NS = 2, 16
NW = NC * NS
CHUNK = 640

B_DIAG = 4096 * 200
B_PROC = 4096 * 50
B_MED = 4096 * 50


def _relu_rows(rows_v, n_rows):
    def body(r, _):
        for c in range(HID // 16):
            sl = pl.ds(c * 16, 16)
            rows_v[r, sl] = jnp.maximum(rows_v[r, sl], 0.0)
        return 0
    lax.fori_loop(0, n_rows, body, 0, unroll=2)


def _branch(idx_hbm, tab_hbm, out_hbm, idx_v, rows_v, sem, wid, total_rows):
    rows_per_w = total_rows // NW
    n_chunks = rows_per_w // CHUNK
    w_base = wid * rows_per_w

    def chunk_body(g, _):
        base = w_base + g * CHUNK
        pltpu.sync_copy(idx_hbm.at[pl.ds(base, CHUNK)], idx_v)
        pltpu.async_copy(tab_hbm.at[idx_v], rows_v, sem).wait()
        _relu_rows(rows_v, CHUNK)
        pltpu.sync_copy(rows_v, out_hbm.at[pl.ds(base, CHUNK)])
        return 0

    lax.fori_loop(0, n_chunks, chunk_body, 0)


@functools.partial(
    pl.kernel,
    out_type=(
        jax.ShapeDtypeStruct((B_DIAG, HID), jnp.float32),
        jax.ShapeDtypeStruct((B_PROC, HID), jnp.float32),
        jax.ShapeDtypeStruct((B_MED, HID), jnp.float32),
    ),
    mesh=plsc.VectorSubcoreMesh(core_axis_name="c", subcore_axis_name="s"),
    scratch_types=[
        pltpu.VMEM((CHUNK,), jnp.int32),
        pltpu.VMEM((CHUNK, HID), jnp.float32),
        pltpu.SemaphoreType.DMA,
    ],
)
def _sc_embed(idx_d, idx_p, idx_m, tab_d, tab_p, tab_m,
              out_d, out_p, out_m, idx_v, rows_v, sem):
    wid = lax.axis_index("s") * NC + lax.axis_index("c")
    _branch(idx_d, tab_d, out_d, idx_v, rows_v, sem, wid, B_DIAG)
    _branch(idx_p, tab_p, out_p, idx_v, rows_v, sem, wid, B_PROC)
    _branch(idx_m, tab_m, out_m, idx_v, rows_v, sem, wid, B_MED)


def _enc_body(x_ref, w_ref, b_ref, o_ref):
    acc = jnp.dot(x_ref[...], w_ref[...], preferred_element_type=jnp.float32)
    o_ref[...] = jnp.maximum(acc + b_ref[...], 0.0)


_enc_call = pl.pallas_call(
    _enc_body,
    out_shape=jax.ShapeDtypeStruct((4096, HID), jnp.float32),
)


@jax.jit
def kernel(encounter, diagnosis, procedure, medication,
           W_enc, b_enc, emb_diag, emb_proc, emb_med):
    out_enc = _enc_call(encounter, W_enc.T, b_enc.reshape(1, HID))
    out_d, out_p, out_m = _sc_embed(
        diagnosis.reshape(-1), procedure.reshape(-1), medication.reshape(-1),
        emb_diag, emb_proc, emb_med)
    return (out_enc, out_d, out_p, out_m)


# speedup vs baseline: 2.7082x; 2.7082x over previous
import functools

import jax
import jax.numpy as jnp
from jax import lax
from jax.experimental import pallas as pl
from jax.experimental.pallas import tpu as pltpu
from jax.experimental.pallas import tpu_sc as plsc

HID = 64
NC, NS = 2, 16
NW = NC * NS
CHUNK = 640

B_DIAG = 4096 * 200
B_PROC = 4096 * 50
B_MED = 4096 * 50


def _relu_rows(rows_v, n_rows):
    def body(r, _):
        for c in range(HID // 16):
            sl = pl.ds(c * 16, 16)
            rows_v[r, sl] = jnp.maximum(rows_v[r, sl], 0.0)
        return 0
    lax.fori_loop(0, n_rows, body, 0, unroll=2)


def _branch(idx_hbm, tab_hbm, out_hbm, idx_v, rows_v, sem, wid, total_rows):
    rows_per_w = total_rows // NW
    n_chunks = rows_per_w // CHUNK
    w_base = wid * rows_per_w

    def chunk_body(g, _):
        base = w_base + g * CHUNK
        pltpu.sync_copy(idx_hbm.at[pl.ds(base, CHUNK)], idx_v)
        pltpu.async_copy(tab_hbm.at[idx_v], rows_v, sem).wait()
        _relu_rows(rows_v, CHUNK)
        pltpu.sync_copy(rows_v, out_hbm.at[pl.ds(base, CHUNK)])
        return 0

    lax.fori_loop(0, n_chunks, chunk_body, 0)


@functools.partial(
    pl.kernel,
    out_type=(
        jax.ShapeDtypeStruct((B_DIAG, HID), jnp.float32),
        jax.ShapeDtypeStruct((B_PROC, HID), jnp.float32),
        jax.ShapeDtypeStruct((B_MED, HID), jnp.float32),
    ),
    mesh=plsc.VectorSubcoreMesh(core_axis_name="c", subcore_axis_name="s"),
    compiler_params=pltpu.CompilerParams(use_tc_tiling_on_sc=False),
    scratch_types=[
        pltpu.VMEM((CHUNK,), jnp.int32),
        pltpu.VMEM((CHUNK, HID), jnp.float32),
        pltpu.SemaphoreType.DMA,
    ],
)
def _sc_embed(idx_d, idx_p, idx_m, tab_d, tab_p, tab_m,
              out_d, out_p, out_m, idx_v, rows_v, sem):
    wid = lax.axis_index("s") * NC + lax.axis_index("c")
    _branch(idx_d, tab_d, out_d, idx_v, rows_v, sem, wid, B_DIAG)
    _branch(idx_p, tab_p, out_p, idx_v, rows_v, sem, wid, B_PROC)
    _branch(idx_m, tab_m, out_m, idx_v, rows_v, sem, wid, B_MED)


def _enc_body(x_ref, w_ref, b_ref, o_ref):
    acc = jnp.dot(x_ref[...], w_ref[...], preferred_element_type=jnp.float32)
    o_ref[...] = jnp.maximum(acc + b_ref[...], 0.0)


_enc_call = pl.pallas_call(
    _enc_body,
    out_shape=jax.ShapeDtypeStruct((4096, HID), jnp.float32),
)


@jax.jit
def kernel(encounter, diagnosis, procedure, medication,
           W_enc, b_enc, emb_diag, emb_proc, emb_med):
    out_enc = _enc_call(encounter, W_enc.T, b_enc.reshape(1, HID))
    out_d, out_p, out_m = _sc_embed(
        diagnosis.reshape(-1), procedure.reshape(-1), medication.reshape(-1),
        emb_diag, emb_proc, emb_med)
    return (out_enc, out_d, out_p, out_m)


# speedup vs baseline: 2.9531x; 1.0904x over previous
import functools

import jax
import jax.numpy as jnp
from jax import lax
from jax.experimental import pallas as pl
from jax.experimental.pallas import tpu as pltpu
from jax.experimental.pallas import tpu_sc as plsc

HID = 64
NC, NS = 2, 16
NW = NC * NS
CHUNK = 320
NSLOT = 4

B_DIAG = 4096 * 200
B_PROC = 4096 * 50
B_MED = 4096 * 50
IDX_MAX = B_DIAG // NW


def _relu_rows(rows_v, s):
    def body(r, _):
        for c in range(HID // 16):
            sl = pl.ds(c * 16, 16)
            rows_v[s, r, sl] = jnp.maximum(rows_v[s, r, sl], 0.0)
        return 0
    lax.fori_loop(0, CHUNK, body, 0, unroll=2)


def _branch(idx_hbm, tab_hbm, out_hbm, idx_v, rows_v, gsem, osem,
            wid, total_rows):
    rows_per_w = total_rows // NW
    n_chunks = rows_per_w // CHUNK
    w_base = wid * rows_per_w

    pltpu.sync_copy(idx_hbm.at[pl.ds(w_base, rows_per_w)],
                    idx_v.at[pl.ds(0, rows_per_w)])

    def gather(g, s):
        return pltpu.make_async_copy(
            tab_hbm.at[idx_v.at[pl.ds(g * CHUNK, CHUNK)]],
            rows_v.at[s], gsem.at[s])

    def out_copy(g, s):
        return pltpu.make_async_copy(
            rows_v.at[s], out_hbm.at[pl.ds(w_base + g * CHUNK, CHUNK)],
            osem.at[s])

    for g in range(NSLOT - 1):
        gather(g, g).start()

    def step(g, _):
        s = lax.rem(g, NSLOT)
        gather(g, s).wait()
        _relu_rows(rows_v, s)
        out_copy(g, s).start()

        @pl.when(g + NSLOT - 1 < n_chunks)
        def _():
            s2 = lax.rem(g + NSLOT - 1, NSLOT)

            @pl.when(g >= 1)
            def _():
                out_copy(g - 1, s2).wait()

            gather(g + NSLOT - 1, s2).start()

        return 0

    lax.fori_loop(0, n_chunks, step, 0)

    for k in range(NSLOT):
        g = n_chunks - NSLOT + k
        out_copy(g, lax.rem(jnp.int32(g), NSLOT)).wait()


@functools.partial(
    pl.kernel,
    out_type=(
        jax.ShapeDtypeStruct((B_DIAG, HID), jnp.float32),
        jax.ShapeDtypeStruct((B_PROC, HID), jnp.float32),
        jax.ShapeDtypeStruct((B_MED, HID), jnp.float32),
    ),
    mesh=plsc.VectorSubcoreMesh(core_axis_name="c", subcore_axis_name="s"),
    compiler_params=pltpu.CompilerParams(use_tc_tiling_on_sc=False),
    scratch_types=[
        pltpu.VMEM((IDX_MAX,), jnp.int32),
        pltpu.VMEM((NSLOT, CHUNK, HID), jnp.float32),
        pltpu.SemaphoreType.DMA((NSLOT,)),
        pltpu.SemaphoreType.DMA((NSLOT,)),
    ],
)
def _sc_embed(idx_d, idx_p, idx_m, tab_d, tab_p, tab_m,
              out_d, out_p, out_m, idx_v, rows_v, gsem, osem):
    wid = lax.axis_index("s") * NC + lax.axis_index("c")
    _branch(idx_d, tab_d, out_d, idx_v, rows_v, gsem, osem, wid, B_DIAG)
    _branch(idx_p, tab_p, out_p, idx_v, rows_v, gsem, osem, wid, B_PROC)
    _branch(idx_m, tab_m, out_m, idx_v, rows_v, gsem, osem, wid, B_MED)


def _enc_body(x_ref, w_ref, b_ref, o_ref):
    acc = jnp.dot(x_ref[...], w_ref[...], preferred_element_type=jnp.float32)
    o_ref[...] = jnp.maximum(acc + b_ref[...], 0.0)


_enc_call = pl.pallas_call(
    _enc_body,
    out_shape=jax.ShapeDtypeStruct((4096, HID), jnp.float32),
)


@jax.jit
def kernel(encounter, diagnosis, procedure, medication,
           W_enc, b_enc, emb_diag, emb_proc, emb_med):
    out_enc = _enc_call(encounter, W_enc.T, b_enc.reshape(1, HID))
    out_d, out_p, out_m = _sc_embed(
        diagnosis.reshape(-1), procedure.reshape(-1), medication.reshape(-1),
        emb_diag, emb_proc, emb_med)
    return (out_enc, out_d, out_p, out_m)
